# baseline (speedup 1.0000x reference)
import functools

import jax
import jax.numpy as jnp
from jax import lax
from jax.experimental import pallas as pl
from jax.experimental.pallas import tpu as pltpu
from jax.experimental.pallas import tpu_sc as plsc

_N = 10000
_NP = 10240
_R = 16
_H = 128
_NB = 8
_T = 12
_E = 160000
_SEQ = 10

_NC = 2
_NS = 16
_NW = _NC * _NS
_K = 128
_CPW = 39
_EPW = _CPW * _K
_NPAIR = (_CPW - 1) // 2
_XBASE = _NW * _EPW
_NX = (_E - _XBASE) // _K
_RPT = _NP // _NS

_NT = 1280
_NG = _NP // _NT


_SC_SCRATCH = [
    pltpu.VMEM((_EPW,), jnp.int32),
    pltpu.VMEM((_K,), jnp.int32),
    pltpu.VMEM((_K,), jnp.int32),
    pltpu.VMEM((_K,), jnp.int32),
    pltpu.VMEM((_K,), jnp.int32),
    pltpu.VMEM((_K, _H), jnp.float32),
    pltpu.VMEM((_K, _H), jnp.float32),
    pltpu.VMEM((_NP,), jnp.float32),
    pltpu.VMEM_SHARED((_NP, _H), jnp.float32),
    pltpu.SemaphoreType.DMA,
    pltpu.SemaphoreType.DMA,
    pltpu.SemaphoreType.DMA,
    pltpu.SemaphoreType.DMA,
]


def _zero_and_edge_phase(table_hbm, idx_hbm, dst_hbm, scratch, s, wid, ebase):
    (idx_st, idx_x, dst_x, dst_va, dst_vb, rows_a, rows_b, cnt_v, acc_sh,
     gsem_a, gsem_b, ssem_a, ssem_b) = scratch

    zero16 = jnp.zeros((16,), jnp.float32)
    ones16 = jnp.ones((16,), jnp.float32)
    mask16 = jnp.ones((16,), jnp.bool_)

    @pl.loop(0, _K * (_H // 16))
    def _(k):
        rows_a[k // (_H // 16), pl.ds((k % (_H // 16)) * 16, 16)] = zero16

    @pl.loop(0, _NP // 16)
    def _(i):
        cnt_v[pl.ds(i * 16, 16)] = zero16

    for kk in range(_RPT // _K):
        pltpu.sync_copy(rows_a, acc_sh.at[pl.ds(s * _RPT + kk * _K, _K)])

    plsc.subcore_barrier()

    base = ebase + wid * _EPW
    pltpu.sync_copy(idx_hbm.at[pl.ds(base, _EPW)], idx_st)

    def cnt_add(dref):
        for j in range(_K // 16):
            dv = dref[pl.ds(j * 16, 16)]
            plsc.addupdate_scatter(cnt_v, [dv], ones16, mask=mask16)

    @pl.when(wid < _NX)
    def _():
        pltpu.sync_copy(idx_hbm.at[pl.ds(ebase + _XBASE + wid * _K, _K)],
                        idx_x)
        pltpu.sync_copy(dst_hbm.at[pl.ds(ebase + _XBASE + wid * _K, _K)],
                        dst_x)
        pltpu.async_copy(table_hbm.at[idx_x], rows_a, gsem_a).wait()
        cnt_add(dst_x)
        pltpu.sync_copy(rows_a, acc_sh.at[dst_x], add=True)

    def start(rows, dstv, gsem, ci):
        pltpu.async_copy(dst_hbm.at[pl.ds(base + ci * _K, _K)], dstv, gsem)
        pltpu.async_copy(
            table_hbm.at[idx_st.at[pl.ds(ci * _K, _K)]], rows, gsem)

    def proc(rows, dstv, gsem, ssem, ci):
        pltpu.make_async_copy(
            dst_hbm.at[pl.ds(base + ci * _K, _K)], dstv, gsem).wait()
        pltpu.make_async_copy(
            table_hbm.at[idx_st.at[pl.ds(ci * _K, _K)]], rows, gsem).wait()
        pltpu.async_copy(rows, acc_sh.at[dstv], ssem, add=True)
        cnt_add(dstv)

    def drain(rows, dstv, ssem):
        pltpu.make_async_copy(rows, acc_sh.at[dstv], ssem).wait()

    start(rows_a, dst_va, gsem_a, 0)
    start(rows_b, dst_vb, gsem_b, 1)

    @pl.loop(0, _NPAIR)
    def _(g):
        proc(rows_a, dst_va, gsem_a, ssem_a, 2 * g)
        proc(rows_b, dst_vb, gsem_b, ssem_b, 2 * g + 1)

        @pl.when(2 * g + 2 < _CPW)
        def _():
            drain(rows_a, dst_va, ssem_a)
            start(rows_a, dst_va, gsem_a, 2 * g + 2)

        @pl.when(2 * g + 3 < _CPW)
        def _():
            drain(rows_b, dst_vb, ssem_b)
            start(rows_b, dst_vb, gsem_b, 2 * g + 3)

    proc(rows_a, dst_va, gsem_a, ssem_a, _CPW - 1)
    drain(rows_a, dst_va, ssem_a)
    drain(rows_b, dst_vb, ssem_b)

    plsc.subcore_barrier()


def _sc_seg_body(table_hbm, idx_hbm, dst_hbm, agg_hbm, cnt_hbm, *scratch):
    c = lax.axis_index("c")
    s = lax.axis_index("s")
    wid = s * _NC + c
    acc_sh = scratch[8]
    cnt_v = scratch[7]

    _zero_and_edge_phase(table_hbm, idx_hbm, dst_hbm, scratch, s, wid, 0)

    pltpu.sync_copy(acc_sh.at[pl.ds(s * _RPT, _RPT)],
                    agg_hbm.at[c, pl.ds(s * _RPT, _RPT)])
    pltpu.sync_copy(cnt_v, cnt_hbm.at[c, s])


def _sc_seg_all_body(table_hbm, idx_hbm, dst_hbm, agg_hbm, cnt_hbm, *scratch):
    c = lax.axis_index("c")
    s = lax.axis_index("s")
    wid = s * _NC + c
    acc_sh = scratch[8]
    cnt_v = scratch[7]

    @pl.loop(0, _T)
    def _(t):
        _zero_and_edge_phase(table_hbm, idx_hbm, dst_hbm, scratch, s, wid,
                             t * _E)
        pltpu.sync_copy(acc_sh.at[pl.ds(s * _RPT, _RPT)],
                        agg_hbm.at[t, c, pl.ds(s * _RPT, _RPT)])
        pltpu.sync_copy(cnt_v, cnt_hbm.at[t, c, s])


@functools.cache
def _get_sc_seg():
  return pl.kernel(
    _sc_seg_body,
    out_type=(
        jax.ShapeDtypeStruct((_NC, _NP, _H), jnp.float32),
        jax.ShapeDtypeStruct((_NC, _NS, _NP), jnp.float32),
    ),
    mesh=plsc.VectorSubcoreMesh(
        core_axis_name="c", subcore_axis_name="s",
        num_cores=_NC, num_subcores=_NS),
    scratch_types=list(_SC_SCRATCH),
    compiler_params=pltpu.CompilerParams(needs_layout_passes=False),
  )


@functools.cache
def _get_sc_seg_all():
  return pl.kernel(
    _sc_seg_all_body,
    out_type=(
        jax.ShapeDtypeStruct((_T, _NC, _NP, _H), jnp.float32),
        jax.ShapeDtypeStruct((_T, _NC, _NS, _NP), jnp.float32),
    ),
    mesh=plsc.VectorSubcoreMesh(
        core_axis_name="c", subcore_axis_name="s",
        num_cores=_NC, num_subcores=_NS),
    scratch_types=list(_SC_SCRATCH),
    compiler_params=pltpu.CompilerParams(needs_layout_passes=False),
  )


def _trans_body(h_ref, w_ref, out_ref):
    h = h_ref[...]
    for r in range(_R):
        out_ref[r] = jnp.dot(h, w_ref[r], preferred_element_type=jnp.float32)


def _trans_table(h, w_dense):
    out = pl.pallas_call(
        _trans_body,
        grid=(_NG,),
        in_specs=[
            pl.BlockSpec((_NT, _H), lambda nb: (nb, 0)),
            pl.BlockSpec((_R, _H, _H), lambda nb: (0, 0, 0)),
        ],
        out_specs=pl.BlockSpec((_R, _NT, _H), lambda nb: (0, nb, 0)),
        out_shape=jax.ShapeDtypeStruct((_R, _NP, _H), jnp.float32),
    )(h, w_dense)
    return out.reshape(_R * _NP, _H)


def _mean_self(agg_ref, cnt_ref):
    a = agg_ref[0] + agg_ref[1]
    cn = jnp.sum(cnt_ref[...].reshape(_NC * _NS, _NT), axis=0)[:, None]
    return a / jnp.maximum(cn, 1.0)


def _combine_relu_trans_body(agg_ref, cnt_ref, h_ref, w_ref, wd_ref,
                             h1_ref, tab_ref):
    m = _mean_self(agg_ref.at[0], cnt_ref.at[0])
    h1 = m + jnp.dot(h_ref[...], w_ref[...], preferred_element_type=jnp.float32)
    h1 = jnp.maximum(h1, 0.0)
    h1_ref[...] = h1
    for r in range(_R):
        tab_ref[r] = jnp.dot(h1, wd_ref[r], preferred_element_type=jnp.float32)


def _combine_relu_trans(agg_all, cnt_all, t, h, w_self, w_dense):
    h1, tab = pl.pallas_call(
        _combine_relu_trans_body,
        grid=(_NG,),
        in_specs=[
            pl.BlockSpec((1, _NC, _NT, _H), lambda nb, _t=t: (_t, 0, nb, 0)),
            pl.BlockSpec((1, _NC, _NS, _NT), lambda nb, _t=t: (_t, 0, 0, nb)),
            pl.BlockSpec((_NT, _H), lambda nb: (nb, 0)),
            pl.BlockSpec((_H, _H), lambda nb: (0, 0)),
            pl.BlockSpec((_R, _H, _H), lambda nb: (0, 0, 0)),
        ],
        out_specs=[
            pl.BlockSpec((_NT, _H), lambda nb: (nb, 0)),
            pl.BlockSpec((_R, _NT, _H), lambda nb: (0, nb, 0)),
        ],
        out_shape=[
            jax.ShapeDtypeStruct((_NP, _H), jnp.float32),
            jax.ShapeDtypeStruct((_R, _NP, _H), jnp.float32),
        ],
    )(agg_all, cnt_all, h, w_self, w_dense)
    return h1, tab.reshape(_R * _NP, _H)


def _combine_max_body(agg_ref, cnt_ref, h_ref, w_ref, out_ref):
    m = _mean_self(agg_ref, cnt_ref.at[0])
    h2 = m + jnp.dot(h_ref[...], w_ref[...], preferred_element_type=jnp.float32)
    rowid = pl.program_id(0) * _NT + lax.broadcasted_iota(jnp.int32, (_NT, 1), 0)
    h2 = jnp.where(rowid < _N, h2, -3e38)
    mb = jnp.broadcast_to(jnp.max(h2, axis=0, keepdims=True), (8, _H))

    @pl.when(pl.program_id(0) == 0)
    def _():
        out_ref[...] = mb

    @pl.when(pl.program_id(0) > 0)
    def _():
        out_ref[...] = jnp.maximum(out_ref[...], mb)


def _combine_max(agg, cnt_all, t, h, w_self):
    out = pl.pallas_call(
        _combine_max_body,
        grid=(_NG,),
        in_specs=[
            pl.BlockSpec((_NC, _NT, _H), lambda nb: (0, nb, 0)),
            pl.BlockSpec((1, _NC, _NS, _NT), lambda nb, _t=t: (_t, 0, 0, nb)),
            pl.BlockSpec((_NT, _H), lambda nb: (nb, 0)),
            pl.BlockSpec((_H, _H), lambda nb: (0, 0)),
        ],
        out_specs=pl.BlockSpec((8, _H), lambda nb: (0, 0)),
        out_shape=jax.ShapeDtypeStruct((8, _H), jnp.float32),
    )(agg, cnt_all, h, w_self)
    return out[0]


_SEQS = []
_LENS = []
_TARGETS_LIST = []
for _i in reversed(range(_T - 1)):
    if _i < _SEQ:
        _SEQS.append(list(range(0, _i + 1)))
        _LENS.append(_i + 1)
    else:
        _SEQS.append(list(range(_i - _SEQ + 1, _i + 1)))
        _LENS.append(_SEQ)
    _TARGETS_LIST.append(_i + 1)
_B = _T - 1


def _tail_body(gh_ref, wih_ref, whh_ref, bih_ref, bhh_ref, lw_ref, lb_ref,
               out_ref):
    gh = gh_ref[...]
    wih = wih_ref[...]
    whh = whh_ref[...]
    bih = bih_ref[...]
    bhh = bhh_ref[...]
    dn = (((1,), (1,)), ((), ()))
    h = jnp.zeros((_B, _H), jnp.float32)
    for j in range(_SEQ):
        rows = []
        for b in range(_B):
            tt = _SEQS[b][j] if j < _LENS[b] else 0
            rows.append(lax.slice(gh, (tt, 0), (tt + 1, _H)))
        x = jnp.concatenate(rows, axis=0)
        gi = lax.dot_general(x, wih, dn,
                             preferred_element_type=jnp.float32) + bih
        gg = lax.dot_general(h, whh, dn,
                             preferred_element_type=jnp.float32) + bhh
        i_r = gi[:, 0:_H]
        i_z = gi[:, _H:2 * _H]
        i_n = gi[:, 2 * _H:3 * _H]
        h_r = gg[:, 0:_H]
        h_z = gg[:, _H:2 * _H]
        h_n = gg[:, 2 * _H:3 * _H]
        r = jax.nn.sigmoid(i_r + h_r)
        z = jax.nn.sigmoid(i_z + h_z)
        n = jnp.tanh(i_n + r * h_n)
        hnew = (1.0 - z) * n + z * h
        assert [j < _LENS[b] for b in range(_B)] == \
               [b < _T - 1 - j or b < 2 for b in range(_B)]
        mask = lax.broadcasted_iota(jnp.int32, (_B, 1), 0) < max(_T - 1 - j, 2)
        h = jnp.where(mask, hnew, h)
    score = lax.dot_general(h, lw_ref[...], dn,
                            preferred_element_type=jnp.float32) + lb_ref[...]
    out_ref[...] = score


def _tail(gh, wih, whh, bih, bhh, lw, lb):
    return pl.pallas_call(
        _tail_body,
        out_shape=jax.ShapeDtypeStruct((_B, _N), jnp.float32),
    )(gh, wih, whh, bih.reshape(1, -1), bhh.reshape(1, -1), lw,
      lb.reshape(1, -1))


def _block_dense(w_rel):
    bi = w_rel.shape[2]
    bo = w_rel.shape[3]
    nb = w_rel.shape[1]
    z = jnp.zeros((w_rel.shape[0], nb, bi, nb, bo), w_rel.dtype)
    bidx = jnp.arange(nb)
    z = z.at[:, bidx, :, bidx, :].set(jnp.transpose(w_rel, (1, 0, 2, 3)))
    return z.reshape(w_rel.shape[0], nb * bi, nb * bo)


def kernel(edges, entity_embed, W_rel1, W_self1, W_rel2, W_self2,
           gru_W_ih, gru_W_hh, gru_b_ih, gru_b_hh, lin_W, lin_b):
    src = edges[:, :, 0]
    rel = edges[:, :, 1] % _R
    dst = edges[:, :, 2].astype(jnp.int32)
    idx_all = (rel * _NP + src).astype(jnp.int32)

    w1d = _block_dense(W_rel1)
    w2d = _block_dense(W_rel2)
    h0 = jnp.pad(entity_embed, ((0, _NP - _N), (0, 0)))
    table1 = _trans_table(h0, w1d)

    sc_seg = _get_sc_seg()
    idx_flat = idx_all.reshape(-1)
    dst_flat = dst.reshape(-1)

    agg1_all, cnt_all = _get_sc_seg_all()(table1, idx_flat, dst_flat)

    gh_list = []
    for t in range(_T):
        h1, table2 = _combine_relu_trans(agg1_all, cnt_all, t, h0, W_self1,
                                         w2d)
        agg2, _cnt2 = sc_seg(table2, idx_all[t], dst[t])
        gh_list.append(_combine_max(agg2, cnt_all, t, h1, W_self2))
    gh = jnp.stack(gh_list, axis=0)
    score = _tail(gh, gru_W_ih, gru_W_hh, gru_b_ih, gru_b_hh, lin_W, lin_b)
    target_index = jnp.arange(_T - 1, 0, -1, dtype=jnp.int32)
    return (score, target_index)


# speedup vs baseline: 1.1650x; 1.1650x over previous
import functools

import jax
import jax.numpy as jnp
from jax import lax
from jax.experimental import pallas as pl
from jax.experimental.pallas import tpu as pltpu
from jax.experimental.pallas import tpu_sc as plsc

_N = 10000
_NP = 10240
_R = 16
_H = 128
_NB = 8
_T = 12
_E = 160000
_SEQ = 10

_NC = 2
_NS = 16
_NW = _NC * _NS
_K = 128
_CPW = 39
_EPW = _CPW * _K
_NPAIR = (_CPW - 1) // 2
_XBASE = _NW * _EPW
_NX = (_E - _XBASE) // _K
_RPT = _NP // _NS

_NT = 1280
_NG = _NP // _NT


_SC_SCRATCH = [
    pltpu.VMEM((_EPW,), jnp.int32),
    pltpu.VMEM((_K,), jnp.int32),
    pltpu.VMEM((_K,), jnp.int32),
    pltpu.VMEM((_K,), jnp.int32),
    pltpu.VMEM((_K,), jnp.int32),
    pltpu.VMEM((_K, _H), jnp.float32),
    pltpu.VMEM((_K, _H), jnp.float32),
    pltpu.VMEM((_NP,), jnp.float32),
    pltpu.VMEM_SHARED((_NP, _H), jnp.float32),
    pltpu.SemaphoreType.DMA,
    pltpu.SemaphoreType.DMA,
    pltpu.SemaphoreType.DMA,
    pltpu.SemaphoreType.DMA,
]


def _zero_and_edge_phase(table_hbm, idx_hbm, dst_hbm, scratch, s, wid, ebase):
    (idx_st, idx_x, dst_x, dst_va, dst_vb, rows_a, rows_b, cnt_v, acc_sh,
     gsem_a, gsem_b, ssem_a, ssem_b) = scratch

    zero16 = jnp.zeros((16,), jnp.float32)
    ones16 = jnp.ones((16,), jnp.float32)
    mask16 = jnp.ones((16,), jnp.bool_)

    @pl.loop(0, _K * (_H // 16))
    def _(k):
        rows_a[k // (_H // 16), pl.ds((k % (_H // 16)) * 16, 16)] = zero16

    @pl.loop(0, _NP // 16)
    def _(i):
        cnt_v[pl.ds(i * 16, 16)] = zero16

    for kk in range(_RPT // _K):
        pltpu.sync_copy(rows_a, acc_sh.at[pl.ds(s * _RPT + kk * _K, _K)])

    plsc.subcore_barrier()

    base = ebase + wid * _EPW
    pltpu.sync_copy(idx_hbm.at[pl.ds(base, _EPW)], idx_st)

    def cnt_add(dref):
        for j in range(_K // 16):
            dv = dref[pl.ds(j * 16, 16)]
            plsc.addupdate_scatter(cnt_v, [dv], ones16, mask=mask16)

    @pl.when(wid < _NX)
    def _():
        pltpu.sync_copy(idx_hbm.at[pl.ds(ebase + _XBASE + wid * _K, _K)],
                        idx_x)
        pltpu.sync_copy(dst_hbm.at[pl.ds(ebase + _XBASE + wid * _K, _K)],
                        dst_x)
        pltpu.async_copy(table_hbm.at[idx_x], rows_a, gsem_a).wait()
        cnt_add(dst_x)
        pltpu.sync_copy(rows_a, acc_sh.at[dst_x], add=True)

    def start(rows, dstv, gsem, ci):
        pltpu.async_copy(dst_hbm.at[pl.ds(base + ci * _K, _K)], dstv, gsem)
        pltpu.async_copy(
            table_hbm.at[idx_st.at[pl.ds(ci * _K, _K)]], rows, gsem)

    def proc(rows, dstv, gsem, ci):
        pltpu.make_async_copy(
            dst_hbm.at[pl.ds(base + ci * _K, _K)], dstv, gsem).wait()
        pltpu.make_async_copy(
            table_hbm.at[idx_st.at[pl.ds(ci * _K, _K)]], rows, gsem).wait()
        cnt_add(dstv)
        pltpu.sync_copy(rows, acc_sh.at[dstv], add=True)

    start(rows_a, dst_va, gsem_a, 0)
    start(rows_b, dst_vb, gsem_b, 1)

    @pl.loop(0, _NPAIR)
    def _(g):
        proc(rows_a, dst_va, gsem_a, 2 * g)

        @pl.when(2 * g + 2 < _CPW)
        def _():
            start(rows_a, dst_va, gsem_a, 2 * g + 2)

        proc(rows_b, dst_vb, gsem_b, 2 * g + 1)

        @pl.when(2 * g + 3 < _CPW)
        def _():
            start(rows_b, dst_vb, gsem_b, 2 * g + 3)

    proc(rows_a, dst_va, gsem_a, _CPW - 1)

    plsc.subcore_barrier()


def _sc_seg_body(table_hbm, idx_hbm, dst_hbm, agg_hbm, cnt_hbm, *scratch):
    c = lax.axis_index("c")
    s = lax.axis_index("s")
    wid = s * _NC + c
    acc_sh = scratch[8]
    cnt_v = scratch[7]

    _zero_and_edge_phase(table_hbm, idx_hbm, dst_hbm, scratch, s, wid, 0)

    pltpu.sync_copy(acc_sh.at[pl.ds(s * _RPT, _RPT)],
                    agg_hbm.at[c, pl.ds(s * _RPT, _RPT)])
    pltpu.sync_copy(cnt_v, cnt_hbm.at[c, s])


def _sc_seg_all_body(table_hbm, idx_hbm, dst_hbm, agg_hbm, cnt_hbm, *scratch):
    c = lax.axis_index("c")
    s = lax.axis_index("s")
    wid = s * _NC + c
    acc_sh = scratch[8]
    cnt_v = scratch[7]

    @pl.loop(0, _T)
    def _(t):
        _zero_and_edge_phase(table_hbm, idx_hbm, dst_hbm, scratch, s, wid,
                             t * _E)
        pltpu.sync_copy(acc_sh.at[pl.ds(s * _RPT, _RPT)],
                        agg_hbm.at[t, c, pl.ds(s * _RPT, _RPT)])
        pltpu.sync_copy(cnt_v, cnt_hbm.at[t, c, s])


@functools.cache
def _get_sc_seg():
  return pl.kernel(
    _sc_seg_body,
    out_type=(
        jax.ShapeDtypeStruct((_NC, _NP, _H), jnp.float32),
        jax.ShapeDtypeStruct((_NC, _NS, _NP), jnp.float32),
    ),
    mesh=plsc.VectorSubcoreMesh(
        core_axis_name="c", subcore_axis_name="s",
        num_cores=_NC, num_subcores=_NS),
    scratch_types=list(_SC_SCRATCH),
    compiler_params=pltpu.CompilerParams(needs_layout_passes=False),
  )


@functools.cache
def _get_sc_seg_all():
  return pl.kernel(
    _sc_seg_all_body,
    out_type=(
        jax.ShapeDtypeStruct((_T, _NC, _NP, _H), jnp.float32),
        jax.ShapeDtypeStruct((_T, _NC, _NS, _NP), jnp.float32),
    ),
    mesh=plsc.VectorSubcoreMesh(
        core_axis_name="c", subcore_axis_name="s",
        num_cores=_NC, num_subcores=_NS),
    scratch_types=list(_SC_SCRATCH),
    compiler_params=pltpu.CompilerParams(needs_layout_passes=False),
  )


def _trans_body(h_ref, w_ref, out_ref):
    h = h_ref[...]
    for r in range(_R):
        out_ref[r] = jnp.dot(h, w_ref[r], preferred_element_type=jnp.float32)


def _trans_table(h, w_dense):
    out = pl.pallas_call(
        _trans_body,
        grid=(_NG,),
        in_specs=[
            pl.BlockSpec((_NT, _H), lambda nb: (nb, 0)),
            pl.BlockSpec((_R, _H, _H), lambda nb: (0, 0, 0)),
        ],
        out_specs=pl.BlockSpec((_R, _NT, _H), lambda nb: (0, nb, 0)),
        out_shape=jax.ShapeDtypeStruct((_R, _NP, _H), jnp.float32),
    )(h, w_dense)
    return out.reshape(_R * _NP, _H)


def _mean_self(agg_ref, cnt_ref):
    a = agg_ref[0] + agg_ref[1]
    cn = jnp.sum(cnt_ref[...].reshape(_NC * _NS, _NT), axis=0)[:, None]
    return a / jnp.maximum(cn, 1.0)


def _combine_relu_trans_body(agg_ref, cnt_ref, h_ref, w_ref, wd_ref,
                             h1_ref, tab_ref):
    m = _mean_self(agg_ref.at[0], cnt_ref.at[0])
    h1 = m + jnp.dot(h_ref[...], w_ref[...], preferred_element_type=jnp.float32)
    h1 = jnp.maximum(h1, 0.0)
    h1_ref[...] = h1
    for r in range(_R):
        tab_ref[r] = jnp.dot(h1, wd_ref[r], preferred_element_type=jnp.float32)


def _combine_relu_trans(agg_all, cnt_all, t, h, w_self, w_dense):
    h1, tab = pl.pallas_call(
        _combine_relu_trans_body,
        grid=(_NG,),
        in_specs=[
            pl.BlockSpec((1, _NC, _NT, _H), lambda nb, _t=t: (_t, 0, nb, 0)),
            pl.BlockSpec((1, _NC, _NS, _NT), lambda nb, _t=t: (_t, 0, 0, nb)),
            pl.BlockSpec((_NT, _H), lambda nb: (nb, 0)),
            pl.BlockSpec((_H, _H), lambda nb: (0, 0)),
            pl.BlockSpec((_R, _H, _H), lambda nb: (0, 0, 0)),
        ],
        out_specs=[
            pl.BlockSpec((_NT, _H), lambda nb: (nb, 0)),
            pl.BlockSpec((_R, _NT, _H), lambda nb: (0, nb, 0)),
        ],
        out_shape=[
            jax.ShapeDtypeStruct((_NP, _H), jnp.float32),
            jax.ShapeDtypeStruct((_R, _NP, _H), jnp.float32),
        ],
    )(agg_all, cnt_all, h, w_self, w_dense)
    return h1, tab.reshape(_R * _NP, _H)


def _combine_max_body(agg_ref, cnt_ref, h_ref, w_ref, out_ref):
    m = _mean_self(agg_ref, cnt_ref.at[0])
    h2 = m + jnp.dot(h_ref[...], w_ref[...], preferred_element_type=jnp.float32)
    rowid = pl.program_id(0) * _NT + lax.broadcasted_iota(jnp.int32, (_NT, 1), 0)
    h2 = jnp.where(rowid < _N, h2, -3e38)
    mb = jnp.broadcast_to(jnp.max(h2, axis=0, keepdims=True), (8, _H))

    @pl.when(pl.program_id(0) == 0)
    def _():
        out_ref[...] = mb

    @pl.when(pl.program_id(0) > 0)
    def _():
        out_ref[...] = jnp.maximum(out_ref[...], mb)


def _combine_max(agg, cnt_all, t, h, w_self):
    out = pl.pallas_call(
        _combine_max_body,
        grid=(_NG,),
        in_specs=[
            pl.BlockSpec((_NC, _NT, _H), lambda nb: (0, nb, 0)),
            pl.BlockSpec((1, _NC, _NS, _NT), lambda nb, _t=t: (_t, 0, 0, nb)),
            pl.BlockSpec((_NT, _H), lambda nb: (nb, 0)),
            pl.BlockSpec((_H, _H), lambda nb: (0, 0)),
        ],
        out_specs=pl.BlockSpec((8, _H), lambda nb: (0, 0)),
        out_shape=jax.ShapeDtypeStruct((8, _H), jnp.float32),
    )(agg, cnt_all, h, w_self)
    return out[0]


_SEQS = []
_LENS = []
_TARGETS_LIST = []
for _i in reversed(range(_T - 1)):
    if _i < _SEQ:
        _SEQS.append(list(range(0, _i + 1)))
        _LENS.append(_i + 1)
    else:
        _SEQS.append(list(range(_i - _SEQ + 1, _i + 1)))
        _LENS.append(_SEQ)
    _TARGETS_LIST.append(_i + 1)
_B = _T - 1


def _tail_body(gh_ref, wih_ref, whh_ref, bih_ref, bhh_ref, lw_ref, lb_ref,
               out_ref):
    gh = gh_ref[...]
    wih = wih_ref[...]
    whh = whh_ref[...]
    bih = bih_ref[...]
    bhh = bhh_ref[...]
    dn = (((1,), (1,)), ((), ()))
    h = jnp.zeros((_B, _H), jnp.float32)
    for j in range(_SEQ):
        rows = []
        for b in range(_B):
            tt = _SEQS[b][j] if j < _LENS[b] else 0
            rows.append(lax.slice(gh, (tt, 0), (tt + 1, _H)))
        x = jnp.concatenate(rows, axis=0)
        gi = lax.dot_general(x, wih, dn,
                             preferred_element_type=jnp.float32) + bih
        gg = lax.dot_general(h, whh, dn,
                             preferred_element_type=jnp.float32) + bhh
        i_r = gi[:, 0:_H]
        i_z = gi[:, _H:2 * _H]
        i_n = gi[:, 2 * _H:3 * _H]
        h_r = gg[:, 0:_H]
        h_z = gg[:, _H:2 * _H]
        h_n = gg[:, 2 * _H:3 * _H]
        r = jax.nn.sigmoid(i_r + h_r)
        z = jax.nn.sigmoid(i_z + h_z)
        n = jnp.tanh(i_n + r * h_n)
        hnew = (1.0 - z) * n + z * h
        assert [j < _LENS[b] for b in range(_B)] == \
               [b < _T - 1 - j or b < 2 for b in range(_B)]
        mask = lax.broadcasted_iota(jnp.int32, (_B, 1), 0) < max(_T - 1 - j, 2)
        h = jnp.where(mask, hnew, h)
    score = lax.dot_general(h, lw_ref[...], dn,
                            preferred_element_type=jnp.float32) + lb_ref[...]
    out_ref[...] = score


def _tail(gh, wih, whh, bih, bhh, lw, lb):
    return pl.pallas_call(
        _tail_body,
        out_shape=jax.ShapeDtypeStruct((_B, _N), jnp.float32),
    )(gh, wih, whh, bih.reshape(1, -1), bhh.reshape(1, -1), lw,
      lb.reshape(1, -1))


def _block_dense(w_rel):
    bi = w_rel.shape[2]
    bo = w_rel.shape[3]
    nb = w_rel.shape[1]
    z = jnp.zeros((w_rel.shape[0], nb, bi, nb, bo), w_rel.dtype)
    bidx = jnp.arange(nb)
    z = z.at[:, bidx, :, bidx, :].set(jnp.transpose(w_rel, (1, 0, 2, 3)))
    return z.reshape(w_rel.shape[0], nb * bi, nb * bo)


def kernel(edges, entity_embed, W_rel1, W_self1, W_rel2, W_self2,
           gru_W_ih, gru_W_hh, gru_b_ih, gru_b_hh, lin_W, lin_b):
    src = edges[:, :, 0]
    rel = edges[:, :, 1] % _R
    dst = edges[:, :, 2].astype(jnp.int32)
    idx_all = (rel * _NP + src).astype(jnp.int32)

    w1d = _block_dense(W_rel1)
    w2d = _block_dense(W_rel2)
    h0 = jnp.pad(entity_embed, ((0, _NP - _N), (0, 0)))
    table1 = _trans_table(h0, w1d)

    sc_seg = _get_sc_seg()
    idx_flat = idx_all.reshape(-1)
    dst_flat = dst.reshape(-1)

    agg1_all, cnt_all = _get_sc_seg_all()(table1, idx_flat, dst_flat)

    gh_list = []
    for t in range(_T):
        h1, table2 = _combine_relu_trans(agg1_all, cnt_all, t, h0, W_self1,
                                         w2d)
        agg2, _cnt2 = sc_seg(table2, idx_all[t], dst[t])
        gh_list.append(_combine_max(agg2, cnt_all, t, h1, W_self2))
    gh = jnp.stack(gh_list, axis=0)
    score = _tail(gh, gru_W_ih, gru_W_hh, gru_b_ih, gru_b_hh, lin_W, lin_b)
    target_index = jnp.arange(_T - 1, 0, -1, dtype=jnp.int32)
    return (score, target_index)


# speedup vs baseline: 1.1922x; 1.0234x over previous
import functools

import jax
import jax.numpy as jnp
from jax import lax
from jax.experimental import pallas as pl
from jax.experimental.pallas import tpu as pltpu
from jax.experimental.pallas import tpu_sc as plsc

_N = 10000
_NP = 10240
_R = 16
_H = 128
_NB = 8
_T = 12
_E = 160000
_SEQ = 10

_NC = 2
_NS = 16
_NW = _NC * _NS
_K = 128
_CPW = 39
_EPW = _CPW * _K
_NPAIR = (_CPW - 1) // 2
_XBASE = _NW * _EPW
_NX = (_E - _XBASE) // _K
_RPT = _NP // _NS

_NT = 1280
_NG = _NP // _NT


_SC_SCRATCH = [
    pltpu.VMEM((_EPW,), jnp.int32),
    pltpu.VMEM((_K,), jnp.int32),
    pltpu.VMEM((_K,), jnp.int32),
    pltpu.VMEM((_K,), jnp.int32),
    pltpu.VMEM((_K,), jnp.int32),
    pltpu.VMEM((_K, _H), jnp.float32),
    pltpu.VMEM((_K, _H), jnp.float32),
    pltpu.VMEM((_NP,), jnp.float32),
    pltpu.VMEM_SHARED((_NP, _H), jnp.float32),
    pltpu.SemaphoreType.DMA,
    pltpu.SemaphoreType.DMA,
    pltpu.SemaphoreType.DMA,
    pltpu.SemaphoreType.DMA,
]


def _zero_and_edge_phase(table_hbm, idx_hbm, dst_hbm, scratch, s, wid, ebase):
    (idx_st, idx_x, dst_x, dst_va, dst_vb, rows_a, rows_b, cnt_v, acc_sh,
     gsem_a, gsem_b, ssem_a, ssem_b) = scratch

    zero16 = jnp.zeros((16,), jnp.float32)
    ones16 = jnp.ones((16,), jnp.float32)
    mask16 = jnp.ones((16,), jnp.bool_)

    base = ebase + wid * _EPW
    pltpu.async_copy(idx_hbm.at[pl.ds(base, _EPW)], idx_st, gsem_a)

    @pl.loop(0, _K * (_H // 16))
    def _(k):
        rows_a[k // (_H // 16), pl.ds((k % (_H // 16)) * 16, 16)] = zero16

    @pl.loop(0, _NP // 16)
    def _(i):
        cnt_v[pl.ds(i * 16, 16)] = zero16

    for kk in range(_RPT // _K):
        pltpu.sync_copy(rows_a, acc_sh.at[pl.ds(s * _RPT + kk * _K, _K)])

    def cnt_add(dref):
        for j in range(_K // 16):
            dv = dref[pl.ds(j * 16, 16)]
            plsc.addupdate_scatter(cnt_v, [dv], ones16, mask=mask16)

    def start(rows, dstv, gsem, ci):
        pltpu.async_copy(dst_hbm.at[pl.ds(base + ci * _K, _K)], dstv, gsem)
        pltpu.async_copy(
            table_hbm.at[idx_st.at[pl.ds(ci * _K, _K)]], rows, gsem)

    def proc(rows, dstv, gsem, ci):
        pltpu.make_async_copy(
            dst_hbm.at[pl.ds(base + ci * _K, _K)], dstv, gsem).wait()
        pltpu.make_async_copy(
            table_hbm.at[idx_st.at[pl.ds(ci * _K, _K)]], rows, gsem).wait()
        cnt_add(dstv)
        pltpu.sync_copy(rows, acc_sh.at[dstv], add=True)

    pltpu.make_async_copy(idx_hbm.at[pl.ds(base, _EPW)], idx_st, gsem_a).wait()
    start(rows_a, dst_va, gsem_a, 0)
    start(rows_b, dst_vb, gsem_b, 1)

    plsc.subcore_barrier()

    @pl.loop(0, _NPAIR)
    def _(g):
        proc(rows_a, dst_va, gsem_a, 2 * g)

        @pl.when(2 * g + 2 < _CPW)
        def _():
            start(rows_a, dst_va, gsem_a, 2 * g + 2)

        proc(rows_b, dst_vb, gsem_b, 2 * g + 1)

        @pl.when(2 * g + 3 < _CPW)
        def _():
            start(rows_b, dst_vb, gsem_b, 2 * g + 3)

    proc(rows_a, dst_va, gsem_a, _CPW - 1)

    @pl.when(wid < _NX)
    def _():
        pltpu.sync_copy(idx_hbm.at[pl.ds(ebase + _XBASE + wid * _K, _K)],
                        idx_x)
        pltpu.sync_copy(dst_hbm.at[pl.ds(ebase + _XBASE + wid * _K, _K)],
                        dst_x)
        pltpu.async_copy(table_hbm.at[idx_x], rows_a, gsem_a).wait()
        cnt_add(dst_x)
        pltpu.sync_copy(rows_a, acc_sh.at[dst_x], add=True)

    plsc.subcore_barrier()


def _sc_seg_body(table_hbm, idx_hbm, dst_hbm, agg_hbm, cnt_hbm, *scratch):
    c = lax.axis_index("c")
    s = lax.axis_index("s")
    wid = s * _NC + c
    acc_sh = scratch[8]
    cnt_v = scratch[7]

    _zero_and_edge_phase(table_hbm, idx_hbm, dst_hbm, scratch, s, wid, 0)

    pltpu.sync_copy(acc_sh.at[pl.ds(s * _RPT, _RPT)],
                    agg_hbm.at[c, pl.ds(s * _RPT, _RPT)])
    pltpu.sync_copy(cnt_v, cnt_hbm.at[c, s])


def _sc_seg_all_body(table_hbm, idx_hbm, dst_hbm, agg_hbm, cnt_hbm, *scratch):
    c = lax.axis_index("c")
    s = lax.axis_index("s")
    wid = s * _NC + c
    acc_sh = scratch[8]
    cnt_v = scratch[7]

    @pl.loop(0, _T)
    def _(t):
        _zero_and_edge_phase(table_hbm, idx_hbm, dst_hbm, scratch, s, wid,
                             t * _E)
        pltpu.sync_copy(acc_sh.at[pl.ds(s * _RPT, _RPT)],
                        agg_hbm.at[t, c, pl.ds(s * _RPT, _RPT)])
        pltpu.sync_copy(cnt_v, cnt_hbm.at[t, c, s])


@functools.cache
def _get_sc_seg():
  return pl.kernel(
    _sc_seg_body,
    out_type=(
        jax.ShapeDtypeStruct((_NC, _NP, _H), jnp.float32),
        jax.ShapeDtypeStruct((_NC, _NS, _NP), jnp.float32),
    ),
    mesh=plsc.VectorSubcoreMesh(
        core_axis_name="c", subcore_axis_name="s",
        num_cores=_NC, num_subcores=_NS),
    scratch_types=list(_SC_SCRATCH),
    compiler_params=pltpu.CompilerParams(needs_layout_passes=False),
  )


@functools.cache
def _get_sc_seg_all():
  return pl.kernel(
    _sc_seg_all_body,
    out_type=(
        jax.ShapeDtypeStruct((_T, _NC, _NP, _H), jnp.float32),
        jax.ShapeDtypeStruct((_T, _NC, _NS, _NP), jnp.float32),
    ),
    mesh=plsc.VectorSubcoreMesh(
        core_axis_name="c", subcore_axis_name="s",
        num_cores=_NC, num_subcores=_NS),
    scratch_types=list(_SC_SCRATCH),
    compiler_params=pltpu.CompilerParams(needs_layout_passes=False),
  )


def _trans_body(h_ref, w_ref, out_ref):
    h = h_ref[...]
    for r in range(_R):
        out_ref[r] = jnp.dot(h, w_ref[r], preferred_element_type=jnp.float32)


def _trans_table(h, w_dense):
    out = pl.pallas_call(
        _trans_body,
        grid=(_NG,),
        in_specs=[
            pl.BlockSpec((_NT, _H), lambda nb: (nb, 0)),
            pl.BlockSpec((_R, _H, _H), lambda nb: (0, 0, 0)),
        ],
        out_specs=pl.BlockSpec((_R, _NT, _H), lambda nb: (0, nb, 0)),
        out_shape=jax.ShapeDtypeStruct((_R, _NP, _H), jnp.float32),
    )(h, w_dense)
    return out.reshape(_R * _NP, _H)


def _mean_self(agg_ref, cnt_ref):
    a = agg_ref[0] + agg_ref[1]
    cn = jnp.sum(cnt_ref[...].reshape(_NC * _NS, _NT), axis=0)[:, None]
    return a / jnp.maximum(cn, 1.0)


def _combine_relu_trans_body(agg_ref, cnt_ref, h_ref, w_ref, wd_ref,
                             h1_ref, tab_ref):
    m = _mean_self(agg_ref.at[0], cnt_ref.at[0])
    h1 = m + jnp.dot(h_ref[...], w_ref[...], preferred_element_type=jnp.float32)
    h1 = jnp.maximum(h1, 0.0)
    h1_ref[...] = h1
    for r in range(_R):
        tab_ref[r] = jnp.dot(h1, wd_ref[r], preferred_element_type=jnp.float32)


def _combine_relu_trans(agg_all, cnt_all, t, h, w_self, w_dense):
    h1, tab = pl.pallas_call(
        _combine_relu_trans_body,
        grid=(_NG,),
        in_specs=[
            pl.BlockSpec((1, _NC, _NT, _H), lambda nb, _t=t: (_t, 0, nb, 0)),
            pl.BlockSpec((1, _NC, _NS, _NT), lambda nb, _t=t: (_t, 0, 0, nb)),
            pl.BlockSpec((_NT, _H), lambda nb: (nb, 0)),
            pl.BlockSpec((_H, _H), lambda nb: (0, 0)),
            pl.BlockSpec((_R, _H, _H), lambda nb: (0, 0, 0)),
        ],
        out_specs=[
            pl.BlockSpec((_NT, _H), lambda nb: (nb, 0)),
            pl.BlockSpec((_R, _NT, _H), lambda nb: (0, nb, 0)),
        ],
        out_shape=[
            jax.ShapeDtypeStruct((_NP, _H), jnp.float32),
            jax.ShapeDtypeStruct((_R, _NP, _H), jnp.float32),
        ],
    )(agg_all, cnt_all, h, w_self, w_dense)
    return h1, tab.reshape(_R * _NP, _H)


def _combine_max_body(agg_ref, cnt_ref, h_ref, w_ref, out_ref):
    m = _mean_self(agg_ref, cnt_ref.at[0])
    h2 = m + jnp.dot(h_ref[...], w_ref[...], preferred_element_type=jnp.float32)
    rowid = pl.program_id(0) * _NT + lax.broadcasted_iota(jnp.int32, (_NT, 1), 0)
    h2 = jnp.where(rowid < _N, h2, -3e38)
    mb = jnp.broadcast_to(jnp.max(h2, axis=0, keepdims=True), (8, _H))

    @pl.when(pl.program_id(0) == 0)
    def _():
        out_ref[...] = mb

    @pl.when(pl.program_id(0) > 0)
    def _():
        out_ref[...] = jnp.maximum(out_ref[...], mb)


def _combine_max(agg, cnt_all, t, h, w_self):
    out = pl.pallas_call(
        _combine_max_body,
        grid=(_NG,),
        in_specs=[
            pl.BlockSpec((_NC, _NT, _H), lambda nb: (0, nb, 0)),
            pl.BlockSpec((1, _NC, _NS, _NT), lambda nb, _t=t: (_t, 0, 0, nb)),
            pl.BlockSpec((_NT, _H), lambda nb: (nb, 0)),
            pl.BlockSpec((_H, _H), lambda nb: (0, 0)),
        ],
        out_specs=pl.BlockSpec((8, _H), lambda nb: (0, 0)),
        out_shape=jax.ShapeDtypeStruct((8, _H), jnp.float32),
    )(agg, cnt_all, h, w_self)
    return out[0]


_SEQS = []
_LENS = []
_TARGETS_LIST = []
for _i in reversed(range(_T - 1)):
    if _i < _SEQ:
        _SEQS.append(list(range(0, _i + 1)))
        _LENS.append(_i + 1)
    else:
        _SEQS.append(list(range(_i - _SEQ + 1, _i + 1)))
        _LENS.append(_SEQ)
    _TARGETS_LIST.append(_i + 1)
_B = _T - 1


def _tail_body(gh_ref, wih_ref, whh_ref, bih_ref, bhh_ref, lw_ref, lb_ref,
               out_ref):
    gh = gh_ref[...]
    wih = wih_ref[...]
    whh = whh_ref[...]
    bih = bih_ref[...]
    bhh = bhh_ref[...]
    dn = (((1,), (1,)), ((), ()))
    h = jnp.zeros((_B, _H), jnp.float32)
    for j in range(_SEQ):
        rows = []
        for b in range(_B):
            tt = _SEQS[b][j] if j < _LENS[b] else 0
            rows.append(lax.slice(gh, (tt, 0), (tt + 1, _H)))
        x = jnp.concatenate(rows, axis=0)
        gi = lax.dot_general(x, wih, dn,
                             preferred_element_type=jnp.float32) + bih
        gg = lax.dot_general(h, whh, dn,
                             preferred_element_type=jnp.float32) + bhh
        i_r = gi[:, 0:_H]
        i_z = gi[:, _H:2 * _H]
        i_n = gi[:, 2 * _H:3 * _H]
        h_r = gg[:, 0:_H]
        h_z = gg[:, _H:2 * _H]
        h_n = gg[:, 2 * _H:3 * _H]
        r = jax.nn.sigmoid(i_r + h_r)
        z = jax.nn.sigmoid(i_z + h_z)
        n = jnp.tanh(i_n + r * h_n)
        hnew = (1.0 - z) * n + z * h
        assert [j < _LENS[b] for b in range(_B)] == \
               [b < _T - 1 - j or b < 2 for b in range(_B)]
        mask = lax.broadcasted_iota(jnp.int32, (_B, 1), 0) < max(_T - 1 - j, 2)
        h = jnp.where(mask, hnew, h)
    score = lax.dot_general(h, lw_ref[...], dn,
                            preferred_element_type=jnp.float32) + lb_ref[...]
    out_ref[...] = score


def _tail(gh, wih, whh, bih, bhh, lw, lb):
    return pl.pallas_call(
        _tail_body,
        out_shape=jax.ShapeDtypeStruct((_B, _N), jnp.float32),
    )(gh, wih, whh, bih.reshape(1, -1), bhh.reshape(1, -1), lw,
      lb.reshape(1, -1))


def _block_dense(w_rel):
    bi = w_rel.shape[2]
    bo = w_rel.shape[3]
    nb = w_rel.shape[1]
    z = jnp.zeros((w_rel.shape[0], nb, bi, nb, bo), w_rel.dtype)
    bidx = jnp.arange(nb)
    z = z.at[:, bidx, :, bidx, :].set(jnp.transpose(w_rel, (1, 0, 2, 3)))
    return z.reshape(w_rel.shape[0], nb * bi, nb * bo)


def kernel(edges, entity_embed, W_rel1, W_self1, W_rel2, W_self2,
           gru_W_ih, gru_W_hh, gru_b_ih, gru_b_hh, lin_W, lin_b):
    src = edges[:, :, 0]
    rel = edges[:, :, 1] % _R
    dst = edges[:, :, 2].astype(jnp.int32)
    idx_all = (rel * _NP + src).astype(jnp.int32)

    w1d = _block_dense(W_rel1)
    w2d = _block_dense(W_rel2)
    h0 = jnp.pad(entity_embed, ((0, _NP - _N), (0, 0)))
    table1 = _trans_table(h0, w1d)

    sc_seg = _get_sc_seg()
    idx_flat = idx_all.reshape(-1)
    dst_flat = dst.reshape(-1)

    agg1_all, cnt_all = _get_sc_seg_all()(table1, idx_flat, dst_flat)

    gh_list = []
    for t in range(_T):
        h1, table2 = _combine_relu_trans(agg1_all, cnt_all, t, h0, W_self1,
                                         w2d)
        agg2, _cnt2 = sc_seg(table2, idx_all[t], dst[t])
        gh_list.append(_combine_max(agg2, cnt_all, t, h1, W_self2))
    gh = jnp.stack(gh_list, axis=0)
    score = _tail(gh, gru_W_ih, gru_W_hh, gru_b_ih, gru_b_hh, lin_W, lin_b)
    target_index = jnp.arange(_T - 1, 0, -1, dtype=jnp.int32)
    return (score, target_index)
